# baseline (speedup 1.0000x reference)
import functools

import jax
import jax.numpy as jnp
from jax import lax
from jax.experimental import pallas as pl
from jax.experimental.pallas import tpu as pltpu
from jax.experimental.pallas import tpu_sc as plsc

N_E = 1024
E_DIM = 256
BETA = 0.25
BT = 1024
N_TOK = 16 * 1024
NB = N_TOK // BT
NB_H = NB // 2

NW = 32
B_PER_W = N_TOK // NW
CHUNK = 64
N_CHUNK = B_PER_W // CHUNK
N_BUF = 4


def _vq_body(final, z_ref, mask_ref, emb_ref, hist_in, loss_in,
             idx_ref, hist_out, loss_ref, perp_ref,
             hist_acc, loss_acc, esq_acc):
    i = pl.program_id(0)
    zb = z_ref[0]
    emb = emb_ref[...]

    @pl.when(i == 0)
    def _init():
        hist_acc[...] = hist_in[...]
        loss_acc[0, 0] = loss_in[0, 0]
        esq_acc[...] = lax.dot_general(
            jnp.ones((8, E_DIM), jnp.float32), emb * emb,
            dimension_numbers=(((1,), (1,)), ((), ())),
            preferred_element_type=jnp.float32,
            precision=lax.Precision.HIGHEST)[0:1]

    zsq = jnp.sum(zb * zb, axis=1, keepdims=True)
    esq = esq_acc[...]
    mm = lax.dot_general(
        zb, emb,
        dimension_numbers=(((1,), (1,)), ((), ())),
        preferred_element_type=jnp.float32)
    d = (zsq + esq) - 2.0 * mm

    dmin = jnp.min(d, axis=1, keepdims=True)
    jidx = lax.broadcasted_iota(jnp.int32, (BT, N_E), 1)
    idx = jnp.min(jnp.where(d == dmin, jidx, N_E), axis=1)
    idx_ref[0, 0, :] = idx

    one_hot = (jidx == idx[:, None]).astype(jnp.bfloat16)
    mb = mask_ref[0, 0, :]
    hist_acc[...] += lax.dot_general(
        jnp.ones((8, BT), jnp.bfloat16), one_hot,
        dimension_numbers=(((1,), (0,)), ((), ())),
        preferred_element_type=jnp.float32)[0:1]
    loss_acc[0, 0] += jnp.sum(mb * dmin[:, 0])

    @pl.when(i == NB_H - 1)
    def _final():
        hist_out[...] = hist_acc[...]
        if final:
            loss_ref[0, 0] = (1.0 + BETA) * loss_acc[0, 0] / (N_TOK * E_DIM)
            e_mean = hist_acc[...] * (1.0 / N_TOK)
            ent = jnp.sum(e_mean * jnp.log(e_mean + 1e-10))
            perp_ref[0, 0] = jnp.exp(-ent)
        else:
            loss_ref[0, 0] = loss_acc[0, 0]
            perp_ref[0, 0] = 0.0


def _tc_argmin(z3, mask3, embedding, hist_in, loss_in, part, final):
    return pl.pallas_call(
        functools.partial(_vq_body, final),
        grid=(NB_H,),
        in_specs=[
            pl.BlockSpec((1, BT, E_DIM), lambda i: (i + part * NB_H, 0, 0)),
            pl.BlockSpec((1, 1, BT), lambda i: (i + part * NB_H, 0, 0)),
            pl.BlockSpec((N_E, E_DIM), lambda i: (0, 0)),
            pl.BlockSpec((1, N_E), lambda i: (0, 0)),
            pl.BlockSpec(memory_space=pltpu.SMEM),
        ],
        out_specs=[
            pl.BlockSpec((1, 1, BT), lambda i: (i, 0, 0)),
            pl.BlockSpec((1, N_E), lambda i: (0, 0)),
            pl.BlockSpec(memory_space=pltpu.SMEM),
            pl.BlockSpec(memory_space=pltpu.SMEM),
        ],
        out_shape=[
            jax.ShapeDtypeStruct((NB_H, 1, BT), jnp.int32),
            jax.ShapeDtypeStruct((1, N_E), jnp.float32),
            jax.ShapeDtypeStruct((1, 1), jnp.float32),
            jax.ShapeDtypeStruct((1, 1), jnp.float32),
        ],
        scratch_shapes=[
            pltpu.VMEM((1, N_E), jnp.float32),
            pltpu.SMEM((1, 1), jnp.float32),
            pltpu.VMEM((1, N_E), jnp.float32),
        ],
    )(z3, mask3, embedding, hist_in, loss_in)


@functools.partial(
    pl.kernel,
    mesh=plsc.VectorSubcoreMesh(core_axis_name="c", subcore_axis_name="s"),
    out_type=jax.ShapeDtypeStruct((N_TOK, E_DIM), jnp.float32),
    scratch_types=(
        [pltpu.VMEM((N_CHUNK, CHUNK), jnp.int32)]
        + [pltpu.VMEM((CHUNK, E_DIM), jnp.float32)] * N_BUF
        + [pltpu.SemaphoreType.DMA] * (2 * N_BUF)
    ),
)
def _sc_gather(emb_hbm, idx_hbm, out_hbm, idx_v, *bufs_and_sems):
    rows = bufs_and_sems[:N_BUF]
    gsem = bufs_and_sems[N_BUF:2 * N_BUF]
    ssem = bufs_and_sems[2 * N_BUF:]
    wid = lax.axis_index("s") * 2 + lax.axis_index("c")
    base = wid * B_PER_W
    pltpu.sync_copy(idx_hbm.at[wid], idx_v)
    gathers = [None] * N_CHUNK
    stores = [None] * N_CHUNK
    for c in range(N_BUF):
        gathers[c] = pltpu.async_copy(emb_hbm.at[idx_v.at[c]],
                                      rows[c], gsem[c])
    for c in range(N_CHUNK):
        b = c % N_BUF
        gathers[c].wait()
        stores[c] = pltpu.async_copy(
            rows[b], out_hbm.at[pl.ds(base + c * CHUNK, CHUNK)], ssem[b])
        nxt = c + N_BUF
        if nxt < N_CHUNK:
            stores[c].wait()
            gathers[nxt] = pltpu.async_copy(emb_hbm.at[idx_v.at[nxt]],
                                            rows[b], gsem[b])
    for c in range(N_CHUNK - N_BUF, N_CHUNK):
        if c >= 0:
            stores[c].wait()


def kernel(z, mask, embedding):
    z3 = z.reshape(NB, BT, E_DIM)
    mask3 = mask.reshape(NB, 1, BT)
    hist0 = jnp.zeros((1, N_E), jnp.float32)
    loss0 = jnp.zeros((1, 1), jnp.float32)
    idx_a, hist_a, loss_a, _ = _tc_argmin(
        z3, mask3, embedding, hist0, loss0, part=0, final=False)
    idx_b, _, loss, perp = _tc_argmin(
        z3, mask3, embedding, hist_a, loss_a, part=1, final=True)
    idx_flat = jnp.concatenate(
        [idx_a.reshape(N_TOK // 2), idx_b.reshape(N_TOK // 2)])
    zq = _sc_gather(embedding, idx_flat.reshape(NW, N_CHUNK, CHUNK))
    return (zq.reshape(z.shape), idx_flat.reshape(N_TOK, 1),
            loss[0, 0], perp[0, 0])


# speedup vs baseline: 1.0399x; 1.0399x over previous
import functools

import jax
import jax.numpy as jnp
from jax import lax
from jax.experimental import pallas as pl
from jax.experimental.pallas import tpu as pltpu
from jax.experimental.pallas import tpu_sc as plsc

N_E = 1024
E_DIM = 256
BETA = 0.25
BT = 1024
N_TOK = 16 * 1024
NB = N_TOK // BT

NW = 32
B_PER_W = N_TOK // NW
CHUNK = 64
N_CHUNK = B_PER_W // CHUNK
N_BUF = 4


def _vq_body(z_ref, mask_ref, emb_ref,
             idx_ref, loss_ref, perp_ref,
             hist_acc, loss_acc, esq_acc):
    i = pl.program_id(0)
    zb = z_ref[0]
    emb = emb_ref[...]

    @pl.when(i == 0)
    def _init():
        hist_acc[...] = jnp.zeros((1, N_E), jnp.float32)
        loss_acc[0, 0] = 0.0
        esq_acc[...] = lax.dot_general(
            jnp.ones((8, E_DIM), jnp.float32), emb * emb,
            dimension_numbers=(((1,), (1,)), ((), ())),
            preferred_element_type=jnp.float32,
            precision=lax.Precision.HIGHEST)[0:1]

    zsq = jnp.sum(zb * zb, axis=1, keepdims=True)
    esq = esq_acc[...]
    mm = lax.dot_general(
        zb, emb,
        dimension_numbers=(((1,), (1,)), ((), ())),
        preferred_element_type=jnp.float32)
    d = (zsq + esq) - 2.0 * mm

    dmin = jnp.min(d, axis=1, keepdims=True)
    jidx = lax.broadcasted_iota(jnp.int32, (BT, N_E), 1)
    idx = jnp.min(jnp.where(d == dmin, jidx, N_E), axis=1)
    idx_ref[0, 0, :] = idx

    one_hot = (jidx == idx[:, None]).astype(jnp.bfloat16)
    mb = mask_ref[0, 0, :]
    hist_acc[...] += lax.dot_general(
        jnp.ones((8, BT), jnp.bfloat16), one_hot,
        dimension_numbers=(((1,), (0,)), ((), ())),
        preferred_element_type=jnp.float32)[0:1]
    loss_acc[0, 0] += jnp.sum(mb * dmin[:, 0])

    @pl.when(i == NB - 1)
    def _final():
        loss_ref[0, 0] = (1.0 + BETA) * loss_acc[0, 0] / (N_TOK * E_DIM)
        e_mean = hist_acc[...] * (1.0 / N_TOK)
        ent = jnp.sum(e_mean * jnp.log(e_mean + 1e-10))
        perp_ref[0, 0] = jnp.exp(-ent)


def _tc_argmin(z3, mask3, embedding):
    return pl.pallas_call(
        _vq_body,
        grid=(NB,),
        in_specs=[
            pl.BlockSpec((1, BT, E_DIM), lambda i: (i, 0, 0)),
            pl.BlockSpec((1, 1, BT), lambda i: (i, 0, 0)),
            pl.BlockSpec((N_E, E_DIM), lambda i: (0, 0)),
        ],
        out_specs=[
            pl.BlockSpec((1, 1, BT), lambda i: (i, 0, 0)),
            pl.BlockSpec(memory_space=pltpu.SMEM),
            pl.BlockSpec(memory_space=pltpu.SMEM),
        ],
        out_shape=[
            jax.ShapeDtypeStruct((NB, 1, BT), jnp.int32),
            jax.ShapeDtypeStruct((1, 1), jnp.float32),
            jax.ShapeDtypeStruct((1, 1), jnp.float32),
        ],
        scratch_shapes=[
            pltpu.VMEM((1, N_E), jnp.float32),
            pltpu.SMEM((1, 1), jnp.float32),
            pltpu.VMEM((1, N_E), jnp.float32),
        ],
    )(z3, mask3, embedding)


@functools.partial(
    pl.kernel,
    mesh=plsc.VectorSubcoreMesh(core_axis_name="c", subcore_axis_name="s"),
    out_type=jax.ShapeDtypeStruct((N_TOK, E_DIM), jnp.float32),
    scratch_types=(
        [pltpu.VMEM((N_CHUNK, CHUNK), jnp.int32)]
        + [pltpu.VMEM((CHUNK, E_DIM), jnp.float32)] * N_BUF
        + [pltpu.SemaphoreType.DMA] * (2 * N_BUF)
    ),
)
def _sc_gather(emb_hbm, idx_hbm, out_hbm, idx_v, *bufs_and_sems):
    rows = bufs_and_sems[:N_BUF]
    gsem = bufs_and_sems[N_BUF:2 * N_BUF]
    ssem = bufs_and_sems[2 * N_BUF:]
    wid = lax.axis_index("s") * 2 + lax.axis_index("c")
    base = wid * B_PER_W
    pltpu.sync_copy(idx_hbm.at[wid], idx_v)
    gathers = [None] * N_CHUNK
    stores = [None] * N_CHUNK
    for c in range(N_BUF):
        gathers[c] = pltpu.async_copy(emb_hbm.at[idx_v.at[c]],
                                      rows[c], gsem[c])
    for c in range(N_CHUNK):
        b = c % N_BUF
        gathers[c].wait()
        stores[c] = pltpu.async_copy(
            rows[b], out_hbm.at[pl.ds(base + c * CHUNK, CHUNK)], ssem[b])
        nxt = c + N_BUF
        if nxt < N_CHUNK:
            stores[c].wait()
            gathers[nxt] = pltpu.async_copy(emb_hbm.at[idx_v.at[nxt]],
                                            rows[b], gsem[b])
    for c in range(N_CHUNK - N_BUF, N_CHUNK):
        if c >= 0:
            stores[c].wait()


def kernel(z, mask, embedding):
    z3 = z.reshape(NB, BT, E_DIM)
    mask3 = mask.reshape(NB, 1, BT)
    idx3, loss, perp = _tc_argmin(z3, mask3, embedding)
    idx_flat = idx3.reshape(N_TOK)
    zq = _sc_gather(embedding, idx_flat.reshape(NW, N_CHUNK, CHUNK))
    return (zq.reshape(z.shape), idx_flat.reshape(N_TOK, 1),
            loss[0, 0], perp[0, 0])


# speedup vs baseline: 1.0849x; 1.0433x over previous
import functools

import jax
import jax.numpy as jnp
from jax import lax
from jax.experimental import pallas as pl
from jax.experimental.pallas import tpu as pltpu
from jax.experimental.pallas import tpu_sc as plsc

N_E = 1024
E_DIM = 256
BETA = 0.25
BT = 1024
N_TOK = 16 * 1024
NB = N_TOK // BT

NW = 32
B_PER_W = N_TOK // NW
CHUNK = 128
N_CHUNK = B_PER_W // CHUNK
N_BUF = 3


def _vq_body(z_ref, mask_ref, emb_ref,
             idx_ref, loss_ref, perp_ref,
             hist_acc, loss_acc, esq_acc):
    i = pl.program_id(0)
    zb = z_ref[0]
    emb = emb_ref[...]

    @pl.when(i == 0)
    def _init():
        hist_acc[...] = jnp.zeros((1, N_E), jnp.float32)
        loss_acc[0, 0] = 0.0
        esq_acc[...] = lax.dot_general(
            jnp.ones((8, E_DIM), jnp.float32), emb * emb,
            dimension_numbers=(((1,), (1,)), ((), ())),
            preferred_element_type=jnp.float32,
            precision=lax.Precision.HIGHEST)[0:1]

    zsq = jnp.sum(zb * zb, axis=1, keepdims=True)
    esq = esq_acc[...]
    mm = lax.dot_general(
        zb, emb,
        dimension_numbers=(((1,), (1,)), ((), ())),
        preferred_element_type=jnp.float32)
    d = (zsq + esq) - 2.0 * mm

    dmin = jnp.min(d, axis=1, keepdims=True)
    jidx = lax.broadcasted_iota(jnp.int32, (BT, N_E), 1)
    idx = jnp.min(jnp.where(d == dmin, jidx, N_E), axis=1)
    idx_ref[0, 0, :] = idx

    one_hot = (jidx == idx[:, None]).astype(jnp.bfloat16)
    mb = mask_ref[0, 0, :]
    hist_acc[...] += lax.dot_general(
        jnp.ones((8, BT), jnp.bfloat16), one_hot,
        dimension_numbers=(((1,), (0,)), ((), ())),
        preferred_element_type=jnp.float32)[0:1]
    loss_acc[0, 0] += jnp.sum(mb * dmin[:, 0])

    @pl.when(i == NB - 1)
    def _final():
        loss_ref[0, 0] = (1.0 + BETA) * loss_acc[0, 0] / (N_TOK * E_DIM)
        e_mean = hist_acc[...] * (1.0 / N_TOK)
        ent = jnp.sum(e_mean * jnp.log(e_mean + 1e-10))
        perp_ref[0, 0] = jnp.exp(-ent)


def _tc_argmin(z3, mask3, embedding):
    return pl.pallas_call(
        _vq_body,
        grid=(NB,),
        in_specs=[
            pl.BlockSpec((1, BT, E_DIM), lambda i: (i, 0, 0)),
            pl.BlockSpec((1, 1, BT), lambda i: (i, 0, 0)),
            pl.BlockSpec((N_E, E_DIM), lambda i: (0, 0)),
        ],
        out_specs=[
            pl.BlockSpec((1, 1, BT), lambda i: (i, 0, 0)),
            pl.BlockSpec(memory_space=pltpu.SMEM),
            pl.BlockSpec(memory_space=pltpu.SMEM),
        ],
        out_shape=[
            jax.ShapeDtypeStruct((NB, 1, BT), jnp.int32),
            jax.ShapeDtypeStruct((1, 1), jnp.float32),
            jax.ShapeDtypeStruct((1, 1), jnp.float32),
        ],
        scratch_shapes=[
            pltpu.VMEM((1, N_E), jnp.float32),
            pltpu.SMEM((1, 1), jnp.float32),
            pltpu.VMEM((1, N_E), jnp.float32),
        ],
    )(z3, mask3, embedding)


@functools.partial(
    pl.kernel,
    mesh=plsc.VectorSubcoreMesh(core_axis_name="c", subcore_axis_name="s"),
    out_type=jax.ShapeDtypeStruct((N_TOK, E_DIM), jnp.float32),
    scratch_types=(
        [pltpu.VMEM((N_CHUNK, CHUNK), jnp.int32)]
        + [pltpu.VMEM((CHUNK, E_DIM), jnp.float32)] * N_BUF
        + [pltpu.SemaphoreType.DMA] * (2 * N_BUF)
    ),
)
def _sc_gather(emb_hbm, idx_hbm, out_hbm, idx_v, *bufs_and_sems):
    rows = bufs_and_sems[:N_BUF]
    gsem = bufs_and_sems[N_BUF:2 * N_BUF]
    ssem = bufs_and_sems[2 * N_BUF:]
    wid = lax.axis_index("s") * 2 + lax.axis_index("c")
    base = wid * B_PER_W
    pltpu.sync_copy(idx_hbm.at[wid], idx_v)
    gathers = [None] * N_CHUNK
    stores = [None] * N_CHUNK
    for c in range(N_BUF):
        gathers[c] = pltpu.async_copy(emb_hbm.at[idx_v.at[c]],
                                      rows[c], gsem[c])
    for c in range(N_CHUNK):
        b = c % N_BUF
        gathers[c].wait()
        stores[c] = pltpu.async_copy(
            rows[b], out_hbm.at[pl.ds(base + c * CHUNK, CHUNK)], ssem[b])
        nxt = c + N_BUF
        if nxt < N_CHUNK:
            stores[c].wait()
            gathers[nxt] = pltpu.async_copy(emb_hbm.at[idx_v.at[nxt]],
                                            rows[b], gsem[b])
    for c in range(N_CHUNK - N_BUF, N_CHUNK):
        if c >= 0:
            stores[c].wait()


def kernel(z, mask, embedding):
    z3 = z.reshape(NB, BT, E_DIM)
    mask3 = mask.reshape(NB, 1, BT)
    idx3, loss, perp = _tc_argmin(z3, mask3, embedding)
    idx_flat = idx3.reshape(N_TOK)
    zq = _sc_gather(embedding, idx_flat.reshape(NW, N_CHUNK, CHUNK))
    return (zq.reshape(z.shape), idx_flat.reshape(N_TOK, 1),
            loss[0, 0], perp[0, 0])
